# baseline (speedup 1.0000x reference)
import functools

import jax
import jax.numpy as jnp
import numpy as np
from jax import lax
from jax.experimental import pallas as pl
from jax.experimental.pallas import tpu as pltpu
from jax.experimental.pallas import tpu_sc as plsc

N = 10000
E = 320000
D = 128
K = 4
L = 2

NC, NS = 2, 16
NW = NC * NS
EPW = E // NW
CH = 40
NCHUNK = EPW // CH
NRCH = N // CH
RCPS = -(-NRCH // NS)

_mesh = plsc.VectorSubcoreMesh(core_axis_name="c", subcore_axis_name="s",
                               num_cores=NC, num_subcores=NS)

_PERM = np.concatenate(
    [np.concatenate([32 * k + 2 * np.arange(16),
                     32 * k + 2 * np.arange(16) + 1]) for k in range(4)])
_UNPERM_M = np.zeros((D, D), np.float32)
_UNPERM_M[np.arange(D), _PERM] = 1.0


@functools.partial(
    pl.kernel,
    out_type=jax.ShapeDtypeStruct((NC, N, D), jnp.float32),
    mesh=_mesh,
    scratch_types=[
        pltpu.VMEM((EPW,), jnp.int32),
        [pltpu.VMEM((CH,), jnp.int32) for _ in range(3)],
        [pltpu.VMEM((CH, D), jnp.float32) for _ in range(3)],
        [pltpu.VMEM((CH, D // 2), jnp.int32) for _ in range(3)],
        pltpu.VMEM_SHARED((N, D), jnp.float32),
        [pltpu.SemaphoreType.DMA for _ in range(3)],
        [pltpu.SemaphoreType.DMA for _ in range(3)],
        [pltpu.SemaphoreType.DMA for _ in range(3)],
        [pltpu.SemaphoreType.DMA for _ in range(3)],
    ],
)
def _sc_aggregate(x_hbm, src_hbm, dst_hbm, ea_hbm, out_hbm,
                  src_v, dst_v, xg_v, ea_v, agg_sh, gsem, esem, ssem, dsem):
    cid = lax.axis_index("c")
    sid = lax.axis_index("s")
    wid = sid * NC + cid

    def _zrow(i, carry):
        for j in range(D // 16):
            xg_v[0][i, pl.ds(j * 16, 16)] = jnp.zeros((16,), jnp.float32)
        return carry
    lax.fori_loop(0, CH, _zrow, 0)
    for z in range(RCPS):
        rc = z * NS + sid

        @pl.when(rc < NRCH)
        def _():
            pltpu.sync_copy(xg_v[0], agg_sh.at[pl.ds(rc * CH, CH)])

    pltpu.sync_copy(src_hbm.at[pl.ds(wid * EPW, EPW)], src_v)

    def _fetch(j, b):
        base = wid * EPW + j * CH
        pltpu.async_copy(dst_hbm.at[pl.ds(base, CH)], dst_v[b], dsem[b])
        pltpu.async_copy(x_hbm.at[src_v.at[pl.ds(j * CH, CH)]],
                         xg_v[b], gsem[b])
        pltpu.async_copy(ea_hbm.at[pl.ds(base, CH)], ea_v[b], esem[b])

    _fetch(0, 0)
    _fetch(1, 1)
    plsc.subcore_barrier()

    def _wait_scatter(b):
        pltpu.make_async_copy(xg_v[b], agg_sh.at[dst_v[b]],
                              ssem[b]).wait()

    def _step(j, b, b2):
        pltpu.make_async_copy(x_hbm.at[src_v.at[pl.ds(0, CH)]], xg_v[b],
                              gsem[b]).wait()
        pltpu.make_async_copy(ea_hbm.at[pl.ds(0, CH)], ea_v[b],
                              esem[b]).wait()

        @pl.when(j >= 1)
        def _():
            _wait_scatter(b2)

        @pl.when(j + 2 < NCHUNK)
        def _():
            _fetch(j + 2, b2)

        pltpu.make_async_copy(dst_hbm.at[pl.ds(0, CH)], dst_v[b],
                              dsem[b]).wait()

        def _row(i, c2):
            for k in range(D // 32):
                e2 = ea_v[b][i, pl.ds(k * 16, 16)]
                elo = lax.bitcast_convert_type(
                    lax.shift_left(e2, 16), jnp.float32)
                ehi = lax.bitcast_convert_type(
                    lax.bitwise_and(e2, jnp.int32(-65536)), jnp.float32)
                slo = pl.ds(k * 32, 16)
                shi = pl.ds(k * 32 + 16, 16)
                xg_v[b][i, slo] = jnp.maximum(xg_v[b][i, slo] + elo, 0.0)
                xg_v[b][i, shi] = jnp.maximum(xg_v[b][i, shi] + ehi, 0.0)
            return c2
        lax.fori_loop(0, CH, _row, 0)
        pltpu.async_copy(xg_v[b], agg_sh.at[dst_v[b]], ssem[b],
                         add=True)

    ntri = (NCHUNK - 1) // 3

    def _tri(t, carry):
        _step(3 * t, 0, 2)
        _step(3 * t + 1, 1, 0)
        _step(3 * t + 2, 2, 1)
        return carry
    lax.fori_loop(0, ntri, _tri, 0)
    _step(NCHUNK - 1, 0, 2)
    _wait_scatter(0)
    plsc.subcore_barrier()

    for z in range(RCPS):
        rc = z * NS + sid

        @pl.when(rc < NRCH)
        def _():
            pltpu.sync_copy(agg_sh.at[pl.ds(rc * CH, CH)],
                            out_hbm.at[cid, pl.ds(rc * CH, CH)])


def _bn(h, g, b):
    m = jnp.mean(h, axis=0, keepdims=True)
    v = jnp.mean((h - m) * (h - m), axis=0, keepdims=True)
    return (h - m) * lax.rsqrt(v + 1e-5) * g + b


def _mask_body(enc_ref, w1_ref, b1_ref, g1_ref, bb1_ref,
               w2_ref, b2_ref, g2_ref, bb2_ref, o_ref):
    h = jnp.dot(enc_ref[...], w1_ref[...],
                preferred_element_type=jnp.float32) + b1_ref[...]
    h = jnp.maximum(_bn(h, g1_ref[...], bb1_ref[...]), 0.0)
    h = jnp.dot(h, w2_ref[...],
                preferred_element_type=jnp.float32) + b2_ref[...]
    h = jnp.maximum(_bn(h, g2_ref[...], bb2_ref[...]), 0.0)
    h = h - jnp.max(h, axis=1, keepdims=True)
    eh = jnp.exp(h)
    o_ref[...] = eh / jnp.sum(eh, axis=1, keepdims=True)


_mask_call = pl.pallas_call(
    _mask_body,
    out_shape=jax.ShapeDtypeStruct((N, K), jnp.float32),
)


def _dense_body(x_ref, a0_ref, a1_ref, xin_ref, m_ref, eps_ref,
                w1_ref, b1_ref, w2_ref, b2_ref, g_ref, bb_ref, um_ref,
                o_ref, *, last, final):
    x = x_ref[...]
    hc = x * eps_ref[...] + (a0_ref[...] + a1_ref[...])
    h = jnp.maximum(
        jnp.dot(hc, w1_ref[...], preferred_element_type=jnp.float32)
        + b1_ref[...], 0.0)
    h = jnp.dot(h, w2_ref[...],
                preferred_element_type=jnp.float32) + b2_ref[...]
    xp = m_ref[...] * h + x
    xn = _bn(xp, g_ref[...], bb_ref[...])
    if last:
        xn = xin_ref[...] + jnp.maximum(xn, 0.0)
    if final:
        xn = jnp.dot(xn, um_ref[...], preferred_element_type=jnp.float32)
    o_ref[...] = xn


_dense_call = {
    (last, final): pl.pallas_call(
        functools.partial(_dense_body, last=last, final=final),
        out_shape=jax.ShapeDtypeStruct((N, D), jnp.float32),
    )
    for last, final in ((False, False), (True, False), (True, True))
}


def kernel(x, edge_index, edge_attr, encoding, cur_layer,
           me_W1, me_b1, me_bn1_g, me_bn1_b, me_W2, me_b2, me_bn2_g,
           me_bn2_b, gine_W1, gine_b1, gine_W2, gine_b2, gine_eps,
           bn_g, bn_b):
    src = edge_index[0]
    dst = edge_index[1]
    ea_bf = lax.bitcast_convert_type(
        edge_attr.astype(jnp.bfloat16).reshape(E, D // 2, 2), jnp.int32)
    x = x[:, _PERM]
    masks = _mask_call(encoding,
                       me_W1, me_b1.reshape(1, -1),
                       me_bn1_g.reshape(1, -1), me_bn1_b.reshape(1, -1),
                       me_W2, me_b2.reshape(1, -1),
                       me_bn2_g.reshape(1, -1), me_bn2_b.reshape(1, -1))
    for l in range(L):
        x_in = x
        for c in range(K):
            agg = _sc_aggregate(x, src, dst, ea_bf)
            epsv = jnp.full((1, D), 1.0 + gine_eps[l, c], jnp.float32)
            last = c == K - 1
            final = last and l == L - 1
            x = _dense_call[(last, final)](
                x, agg[0], agg[1], x_in, masks[:, c:c + 1], epsv,
                gine_W1[l, c][_PERM, :], gine_b1[l, c].reshape(1, -1),
                gine_W2[l, c][:, _PERM], gine_b2[l, c][_PERM].reshape(1, -1),
                bn_g[l, c][_PERM].reshape(1, -1),
                bn_b[l, c][_PERM].reshape(1, -1),
                jnp.asarray(_UNPERM_M))
    return x


# speedup vs baseline: 1.4553x; 1.4553x over previous
import functools

import jax
import jax.numpy as jnp
import numpy as np
from jax import lax
from jax.experimental import pallas as pl
from jax.experimental.pallas import tpu as pltpu
from jax.experimental.pallas import tpu_sc as plsc

N = 10000
E = 320000
D = 128
K = 4
L = 2

NC, NS = 2, 16
NW = NC * NS
EPW = E // NW
CH = 40
NCHUNK = EPW // CH
NRCH = N // CH
RCPS = -(-NRCH // NS)

_mesh = plsc.VectorSubcoreMesh(core_axis_name="c", subcore_axis_name="s",
                               num_cores=NC, num_subcores=NS)


@functools.partial(
    pl.kernel,
    out_type=jax.ShapeDtypeStruct((NC, N, D), jnp.float32),
    mesh=_mesh,
    scratch_types=[
        pltpu.VMEM((EPW,), jnp.int32),
        [pltpu.VMEM((CH,), jnp.int32) for _ in range(3)],
        [pltpu.VMEM((CH, D), jnp.float32) for _ in range(3)],
        [pltpu.VMEM((CH, D // 2), jnp.int32) for _ in range(3)],
        pltpu.VMEM_SHARED((N, D), jnp.float32),
        [pltpu.SemaphoreType.DMA for _ in range(3)],
        [pltpu.SemaphoreType.DMA for _ in range(3)],
        [pltpu.SemaphoreType.DMA for _ in range(3)],
        [pltpu.SemaphoreType.DMA for _ in range(3)],
    ],
)
def _sc_aggregate(x_hbm, src_hbm, dst_hbm, ea_hbm, out_hbm,
                  src_v, dst_v, xg_v, ea_v, agg_sh, gsem, esem, ssem, dsem):
    cid = lax.axis_index("c")
    sid = lax.axis_index("s")
    wid = sid * NC + cid

    def _zrow(i, carry):
        for j in range(D // 16):
            xg_v[0][i, pl.ds(j * 16, 16)] = jnp.zeros((16,), jnp.float32)
        return carry
    lax.fori_loop(0, CH, _zrow, 0)
    for z in range(RCPS):
        rc = z * NS + sid

        @pl.when(rc < NRCH)
        def _():
            pltpu.sync_copy(xg_v[0], agg_sh.at[pl.ds(rc * CH, CH)])

    pltpu.sync_copy(src_hbm.at[pl.ds(wid * EPW, EPW)], src_v)

    def _fetch(j, b):
        base = wid * EPW + j * CH
        pltpu.async_copy(dst_hbm.at[pl.ds(base, CH)], dst_v[b], dsem[b])
        pltpu.async_copy(x_hbm.at[src_v.at[pl.ds(j * CH, CH)]],
                         xg_v[b], gsem[b])
        pltpu.async_copy(ea_hbm.at[pl.ds(base, CH)], ea_v[b], esem[b])

    _fetch(0, 0)
    _fetch(1, 1)
    plsc.subcore_barrier()

    def _wait_scatter(b):
        pltpu.make_async_copy(xg_v[b], agg_sh.at[dst_v[b]],
                              ssem[b]).wait()

    def _step(j, b, b2):
        pltpu.make_async_copy(x_hbm.at[src_v.at[pl.ds(0, CH)]], xg_v[b],
                              gsem[b]).wait()
        pltpu.make_async_copy(ea_hbm.at[pl.ds(0, CH)], ea_v[b],
                              esem[b]).wait()

        @pl.when(j >= 1)
        def _():
            _wait_scatter(b2)

        @pl.when(j + 2 < NCHUNK)
        def _():
            _fetch(j + 2, b2)

        pltpu.make_async_copy(dst_hbm.at[pl.ds(0, CH)], dst_v[b],
                              dsem[b]).wait()

        def _row(i, c2):
            for k in range(D // 32):
                e2 = ea_v[b][i, pl.ds(k * 16, 16)]
                elo = lax.bitcast_convert_type(
                    lax.shift_left(e2, 16), jnp.float32)
                ehi = lax.bitcast_convert_type(
                    lax.bitwise_and(e2, jnp.int32(-65536)), jnp.float32)
                slo = pl.ds(k * 16, 16)
                shi = pl.ds(64 + k * 16, 16)
                xg_v[b][i, slo] = jnp.maximum(xg_v[b][i, slo] + elo, 0.0)
                xg_v[b][i, shi] = jnp.maximum(xg_v[b][i, shi] + ehi, 0.0)
            return c2
        lax.fori_loop(0, CH, _row, 0)
        pltpu.async_copy(xg_v[b], agg_sh.at[dst_v[b]], ssem[b],
                         add=True)

    ntri = (NCHUNK - 1) // 3

    def _tri(t, carry):
        _step(3 * t, 0, 2)
        _step(3 * t + 1, 1, 0)
        _step(3 * t + 2, 2, 1)
        return carry
    lax.fori_loop(0, ntri, _tri, 0)
    _step(NCHUNK - 1, 0, 2)
    _wait_scatter(0)
    plsc.subcore_barrier()

    for z in range(RCPS):
        rc = z * NS + sid

        @pl.when(rc < NRCH)
        def _():
            pltpu.sync_copy(agg_sh.at[pl.ds(rc * CH, CH)],
                            out_hbm.at[cid, pl.ds(rc * CH, CH)])


def _bn(h, g, b):
    m = jnp.mean(h, axis=0, keepdims=True)
    v = jnp.mean((h - m) * (h - m), axis=0, keepdims=True)
    return (h - m) * lax.rsqrt(v + 1e-5) * g + b


def _mask_body(enc_ref, w1_ref, b1_ref, g1_ref, bb1_ref,
               w2_ref, b2_ref, g2_ref, bb2_ref, o_ref):
    h = jnp.dot(enc_ref[...], w1_ref[...],
                preferred_element_type=jnp.float32) + b1_ref[...]
    h = jnp.maximum(_bn(h, g1_ref[...], bb1_ref[...]), 0.0)
    h = jnp.dot(h, w2_ref[...],
                preferred_element_type=jnp.float32) + b2_ref[...]
    h = jnp.maximum(_bn(h, g2_ref[...], bb2_ref[...]), 0.0)
    h = h - jnp.max(h, axis=1, keepdims=True)
    eh = jnp.exp(h)
    o_ref[...] = eh / jnp.sum(eh, axis=1, keepdims=True)


_mask_call = pl.pallas_call(
    _mask_body,
    out_shape=jax.ShapeDtypeStruct((N, K), jnp.float32),
)


def _dense_body(x_ref, a0_ref, a1_ref, xin_ref, m_ref, eps_ref,
                w1_ref, b1_ref, w2_ref, b2_ref, g_ref, bb_ref, o_ref,
                *, last):
    x = x_ref[...]
    hc = x * eps_ref[...] + (a0_ref[...] + a1_ref[...])
    h = jnp.maximum(
        jnp.dot(hc, w1_ref[...], preferred_element_type=jnp.float32)
        + b1_ref[...], 0.0)
    h = jnp.dot(h, w2_ref[...],
                preferred_element_type=jnp.float32) + b2_ref[...]
    xp = m_ref[...] * h + x
    xn = _bn(xp, g_ref[...], bb_ref[...])
    if last:
        xn = xin_ref[...] + jnp.maximum(xn, 0.0)
    o_ref[...] = xn


_dense_call = {
    last: pl.pallas_call(
        functools.partial(_dense_body, last=last),
        out_shape=jax.ShapeDtypeStruct((N, D), jnp.float32),
    )
    for last in (False, True)
}

_PACKB = 4000


def _pack_body(x_ref, o_ref):
    a = lax.bitcast_convert_type(
        x_ref[:, :D // 2].astype(jnp.bfloat16), jnp.uint16)
    b = lax.bitcast_convert_type(
        x_ref[:, D // 2:].astype(jnp.bfloat16), jnp.uint16)
    o_ref[...] = lax.shift_left(b.astype(jnp.int32), 16) | a.astype(
        jnp.int32)


_pack_call = pl.pallas_call(
    _pack_body,
    grid=(E // _PACKB,),
    in_specs=[pl.BlockSpec((_PACKB, D), lambda i: (i, 0))],
    out_specs=pl.BlockSpec((_PACKB, D // 2), lambda i: (i, 0)),
    out_shape=jax.ShapeDtypeStruct((E, D // 2), jnp.int32),
)


def kernel(x, edge_index, edge_attr, encoding, cur_layer,
           me_W1, me_b1, me_bn1_g, me_bn1_b, me_W2, me_b2, me_bn2_g,
           me_bn2_b, gine_W1, gine_b1, gine_W2, gine_b2, gine_eps,
           bn_g, bn_b):
    src = edge_index[0]
    dst = edge_index[1]
    ea_bf = _pack_call(edge_attr)
    masks = _mask_call(encoding,
                       me_W1, me_b1.reshape(1, -1),
                       me_bn1_g.reshape(1, -1), me_bn1_b.reshape(1, -1),
                       me_W2, me_b2.reshape(1, -1),
                       me_bn2_g.reshape(1, -1), me_bn2_b.reshape(1, -1))
    for l in range(L):
        x_in = x
        for c in range(K):
            agg = _sc_aggregate(x, src, dst, ea_bf)
            epsv = jnp.full((1, D), 1.0 + gine_eps[l, c], jnp.float32)
            x = _dense_call[c == K - 1](
                x, agg[0], agg[1], x_in, masks[:, c:c + 1], epsv,
                gine_W1[l, c], gine_b1[l, c].reshape(1, -1),
                gine_W2[l, c], gine_b2[l, c].reshape(1, -1),
                bn_g[l, c].reshape(1, -1), bn_b[l, c].reshape(1, -1))
    return x


# speedup vs baseline: 1.5450x; 1.0617x over previous
import functools

import jax
import jax.numpy as jnp
import numpy as np
from jax import lax
from jax.experimental import pallas as pl
from jax.experimental.pallas import tpu as pltpu
from jax.experimental.pallas import tpu_sc as plsc

N = 10000
E = 320000
D = 128
K = 4
L = 2

NC, NS = 2, 16
NW = NC * NS
EPW = E // NW
CH = 40
NCHUNK = EPW // CH
NRCH = N // CH
RCPS = -(-NRCH // NS)

_mesh = plsc.VectorSubcoreMesh(core_axis_name="c", subcore_axis_name="s",
                               num_cores=NC, num_subcores=NS)


@functools.partial(
    pl.kernel,
    out_type=jax.ShapeDtypeStruct((NC, N, D), jnp.float32),
    mesh=_mesh,
    scratch_types=[
        pltpu.VMEM((EPW,), jnp.int32),
        [pltpu.VMEM((CH,), jnp.int32) for _ in range(3)],
        [pltpu.VMEM((CH, D), jnp.float32) for _ in range(3)],
        [pltpu.VMEM((CH, D), jnp.float32) for _ in range(3)],
        pltpu.VMEM_SHARED((N, D), jnp.float32),
        [pltpu.SemaphoreType.DMA for _ in range(3)],
        [pltpu.SemaphoreType.DMA for _ in range(3)],
        [pltpu.SemaphoreType.DMA for _ in range(3)],
        [pltpu.SemaphoreType.DMA for _ in range(3)],
    ],
)
def _sc_aggregate(x_hbm, src_hbm, dst_hbm, ea_hbm, out_hbm,
                  src_v, dst_v, xg_v, ea_v, agg_sh, gsem, esem, ssem, dsem):
    cid = lax.axis_index("c")
    sid = lax.axis_index("s")
    wid = sid * NC + cid

    pltpu.sync_copy(src_hbm.at[pl.ds(wid * EPW, EPW)], src_v)

    def _fetch(j, b):
        base = wid * EPW + j * CH
        pltpu.async_copy(dst_hbm.at[pl.ds(base, CH)], dst_v[b], dsem[b])
        pltpu.async_copy(x_hbm.at[src_v.at[pl.ds(j * CH, CH)]],
                         xg_v[b], gsem[b])
        pltpu.async_copy(ea_hbm.at[pl.ds(base, CH)], ea_v[b], esem[b])

    _fetch(0, 0)
    _fetch(1, 1)

    def _zrow(i, carry):
        for j in range(D // 16):
            ea_v[2][i, pl.ds(j * 16, 16)] = jnp.zeros((16,), jnp.float32)
        return carry
    lax.fori_loop(0, CH, _zrow, 0)
    for z in range(RCPS):
        rc = z * NS + sid

        @pl.when(rc < NRCH)
        def _():
            pltpu.async_copy(ea_v[2], agg_sh.at[pl.ds(rc * CH, CH)],
                             ssem[0])
    for z in range(RCPS):
        rc = z * NS + sid

        @pl.when(rc < NRCH)
        def _():
            pltpu.make_async_copy(ea_v[2], agg_sh.at[pl.ds(0, CH)],
                                  ssem[0]).wait()
    plsc.subcore_barrier()

    def _wait_scatter(b):
        pltpu.make_async_copy(xg_v[b], agg_sh.at[dst_v[b]],
                              ssem[b]).wait()

    def _step(j, b, b2):
        pltpu.make_async_copy(x_hbm.at[src_v.at[pl.ds(0, CH)]], xg_v[b],
                              gsem[b]).wait()
        pltpu.make_async_copy(ea_hbm.at[pl.ds(0, CH)], ea_v[b],
                              esem[b]).wait()

        @pl.when(j >= 1)
        def _():
            _wait_scatter(b2)

        @pl.when(j + 2 < NCHUNK)
        def _():
            _fetch(j + 2, b2)

        pltpu.make_async_copy(dst_hbm.at[pl.ds(0, CH)], dst_v[b],
                              dsem[b]).wait()

        def _row(i, c2):
            for jj in range(D // 16):
                sl = pl.ds(jj * 16, 16)
                xg_v[b][i, sl] = jnp.maximum(
                    xg_v[b][i, sl] + ea_v[b][i, sl], 0.0)
            return c2
        lax.fori_loop(0, CH, _row, 0)
        pltpu.async_copy(xg_v[b], agg_sh.at[dst_v[b]], ssem[b],
                         add=True)

    ntri = (NCHUNK - 1) // 3

    def _tri(t, carry):
        _step(3 * t, 0, 2)
        _step(3 * t + 1, 1, 0)
        _step(3 * t + 2, 2, 1)
        return carry
    lax.fori_loop(0, ntri, _tri, 0)
    _step(NCHUNK - 1, 0, 2)
    _wait_scatter(0)
    plsc.subcore_barrier()

    for z in range(RCPS):
        rc = z * NS + sid

        @pl.when(rc < NRCH)
        def _():
            pltpu.async_copy(agg_sh.at[pl.ds(rc * CH, CH)],
                             out_hbm.at[cid, pl.ds(rc * CH, CH)], ssem[0])
    for z in range(RCPS):
        rc = z * NS + sid

        @pl.when(rc < NRCH)
        def _():
            pltpu.make_async_copy(agg_sh.at[pl.ds(0, CH)],
                                  out_hbm.at[cid, pl.ds(0, CH)],
                                  ssem[0]).wait()


def _bn(h, g, b):
    m = jnp.mean(h, axis=0, keepdims=True)
    v = jnp.mean((h - m) * (h - m), axis=0, keepdims=True)
    return (h - m) * lax.rsqrt(v + 1e-5) * g + b


def _mask_body(enc_ref, w1_ref, b1_ref, g1_ref, bb1_ref,
               w2_ref, b2_ref, g2_ref, bb2_ref, o_ref):
    h = jnp.dot(enc_ref[...], w1_ref[...],
                preferred_element_type=jnp.float32) + b1_ref[...]
    h = jnp.maximum(_bn(h, g1_ref[...], bb1_ref[...]), 0.0)
    h = jnp.dot(h, w2_ref[...],
                preferred_element_type=jnp.float32) + b2_ref[...]
    h = jnp.maximum(_bn(h, g2_ref[...], bb2_ref[...]), 0.0)
    h = h - jnp.max(h, axis=1, keepdims=True)
    eh = jnp.exp(h)
    o_ref[...] = eh / jnp.sum(eh, axis=1, keepdims=True)


_mask_call = pl.pallas_call(
    _mask_body,
    out_shape=jax.ShapeDtypeStruct((N, K), jnp.float32),
)


def _dense_body(x_ref, a0_ref, a1_ref, xin_ref, m_ref, eps_ref,
                w1_ref, b1_ref, w2_ref, b2_ref, g_ref, bb_ref, o_ref,
                *, last):
    x = x_ref[...]
    hc = x * eps_ref[...] + (a0_ref[...] + a1_ref[...])
    h = jnp.maximum(
        jnp.dot(hc, w1_ref[...], preferred_element_type=jnp.float32)
        + b1_ref[...], 0.0)
    h = jnp.dot(h, w2_ref[...],
                preferred_element_type=jnp.float32) + b2_ref[...]
    xp = m_ref[...] * h + x
    xn = _bn(xp, g_ref[...], bb_ref[...])
    if last:
        xn = xin_ref[...] + jnp.maximum(xn, 0.0)
    o_ref[...] = xn


_dense_call = {
    last: pl.pallas_call(
        functools.partial(_dense_body, last=last),
        out_shape=jax.ShapeDtypeStruct((N, D), jnp.float32),
    )
    for last in (False, True)
}

def kernel(x, edge_index, edge_attr, encoding, cur_layer,
           me_W1, me_b1, me_bn1_g, me_bn1_b, me_W2, me_b2, me_bn2_g,
           me_bn2_b, gine_W1, gine_b1, gine_W2, gine_b2, gine_eps,
           bn_g, bn_b):
    src = edge_index[0]
    dst = edge_index[1]
    masks = _mask_call(encoding,
                       me_W1, me_b1.reshape(1, -1),
                       me_bn1_g.reshape(1, -1), me_bn1_b.reshape(1, -1),
                       me_W2, me_b2.reshape(1, -1),
                       me_bn2_g.reshape(1, -1), me_bn2_b.reshape(1, -1))
    for l in range(L):
        x_in = x
        for c in range(K):
            agg = _sc_aggregate(x, src, dst, edge_attr)
            epsv = jnp.full((1, D), 1.0 + gine_eps[l, c], jnp.float32)
            x = _dense_call[c == K - 1](
                x, agg[0], agg[1], x_in, masks[:, c:c + 1], epsv,
                gine_W1[l, c], gine_b1[l, c].reshape(1, -1),
                gine_W2[l, c], gine_b2[l, c].reshape(1, -1),
                bn_g[l, c].reshape(1, -1), bn_b[l, c].reshape(1, -1))
    return x
